# baseline (speedup 1.0000x reference)
import dataclasses

import jax
import jax.numpy as jnp
from jax import lax
from jax.experimental import pallas as pl
from jax.experimental.pallas import tpu as pltpu
from jax.experimental.pallas import tpu_sc as plsc

_SC_PARAMS = pltpu.CompilerParams()
if "needs_layout_passes" in pltpu.CompilerParams.__dataclass_fields__:
    _SC_PARAMS = dataclasses.replace(_SC_PARAMS, needs_layout_passes=False)
if "use_tc_tiling_on_sc" in pltpu.CompilerParams.__dataclass_fields__:
    _SC_PARAMS = dataclasses.replace(_SC_PARAMS, use_tc_tiling_on_sc=False)

N = 10000
E = 320000
D = 128
F = 64

NC = 2
NS = 16
NW = NC * NS
EPW = E // NW
STRIP = 80
NSTR = EPW // STRIP
GRP = 5
NGRP = NSTR // GRP
CPY = 624
TAIL = N - NS * CPY
NP = 10240
DCPY = NP // NS


def _tc_first_body(x_ref, w_ref, as_ref, ad_ref, h_ref, es_ref, ed_ref, m_ref):
    h = jnp.dot(x_ref[...], w_ref[...], preferred_element_type=jnp.float32)
    h_ref[...] = h
    es = jnp.sum(h * as_ref[...], axis=1, keepdims=True)
    ed = jnp.sum(h * ad_ref[...], axis=1, keepdims=True)
    es_ref[...] = es
    ed_ref[...] = ed
    m = jnp.maximum(jnp.max(es) + jnp.max(ed), 0.0)
    m_ref[...] = jnp.full((8, 128), m, jnp.float32)


def _tc_mid_body(agg_ref, den_ref, b_ref, w_ref, as_ref, ad_ref,
                 h_ref, es_ref, ed_ref, m_ref):
    a = agg_ref[0] + agg_ref[1]
    dn = den_ref[0, :N] + den_ref[1, :N]
    x = jnp.maximum(a / (dn + 1e-30) + b_ref[...], 0.0)
    h = jnp.dot(x, w_ref[...], preferred_element_type=jnp.float32)
    h_ref[...] = h
    es = jnp.sum(h * as_ref[...], axis=1, keepdims=True)
    ed = jnp.sum(h * ad_ref[...], axis=1, keepdims=True)
    es_ref[...] = es
    ed_ref[...] = ed
    m = jnp.maximum(jnp.max(es) + jnp.max(ed), 0.0)
    m_ref[...] = jnp.full((8, 128), m, jnp.float32)


def _tc_head_body(agg_ref, den_ref, b_ref, fcw_ref, fcb_ref, ow_ref, ob_ref,
                  out_ref):
    a = agg_ref[0] + agg_ref[1]
    dn = den_ref[0, :N] + den_ref[1, :N]
    x = jnp.maximum(a / (dn + 1e-30) + b_ref[...], 0.0)
    pooled = jnp.mean(x, axis=0, keepdims=True)
    hfc = jnp.maximum(
        jnp.dot(pooled, fcw_ref[...], preferred_element_type=jnp.float32)
        + fcb_ref[...], 0.0)
    out_ref[...] = jax.nn.sigmoid(
        jnp.dot(hfc, ow_ref[...], preferred_element_type=jnp.float32)
        + ob_ref[...])


_LAYER_OUT = [
    jax.ShapeDtypeStruct((N, F), jnp.float32),
    jax.ShapeDtypeStruct((N, 1), jnp.float32),
    jax.ShapeDtypeStruct((N, 1), jnp.float32),
    jax.ShapeDtypeStruct((8, 128), jnp.float32),
]

_tc_first = pl.pallas_call(_tc_first_body, out_shape=_LAYER_OUT)
_tc_mid = pl.pallas_call(_tc_mid_body, out_shape=_LAYER_OUT)
_tc_head = pl.pallas_call(
    _tc_head_body, out_shape=jax.ShapeDtypeStruct((1, 1), jnp.float32))


def _make_sc_edge(want_ex):
    outs = [
        jax.ShapeDtypeStruct((NC, N, F), jnp.float32),
        jax.ShapeDtypeStruct((NC, NP), jnp.float32),
    ]
    if want_ex:
        outs.append(jax.ShapeDtypeStruct((NW, NGRP, GRP * STRIP), jnp.float32))
    scratch = [
        pltpu.VMEM((N,), jnp.float32),
        pltpu.VMEM((N,), jnp.float32),
        pltpu.VMEM((NGRP, GRP * STRIP), jnp.int32),
        pltpu.VMEM((NGRP, GRP * STRIP), jnp.int32),
        pltpu.VMEM((16,), jnp.float32),
        pltpu.VMEM((GRP * STRIP,), jnp.float32),
        pltpu.VMEM((GRP * STRIP, F), jnp.float32),
        pltpu.VMEM_SHARED((N, F), jnp.float32),
        pltpu.VMEM_SHARED((NP,), jnp.float32),
    ] + [pltpu.SemaphoreType.DMA] * 2
    mesh = plsc.VectorSubcoreMesh(core_axis_name="c", subcore_axis_name="s")

    def body(h_hbm, es_hbm, ed_hbm, src_hbm, dst_hbm, m_hbm, z64_hbm, z1_hbm,
             agg_hbm, den_hbm, *rest):
        if want_ex:
            ex_hbm, *rest = rest
        (es_v, ed_v, src_v, dst_v, m_v, ex_v, rows_v, sout, sden,
         gsem, ssem) = rest
        c = lax.axis_index("c")
        s = lax.axis_index("s")
        wid = c * NS + s
        pltpu.sync_copy(es_hbm, es_v)
        pltpu.sync_copy(ed_hbm, ed_v)
        pltpu.sync_copy(src_hbm.at[wid], src_v)
        pltpu.sync_copy(dst_hbm.at[wid], dst_v)
        pltpu.sync_copy(m_hbm.at[0, pl.ds(0, 16)], m_v)
        r0 = s * CPY
        d0 = s * DCPY
        pltpu.sync_copy(z64_hbm.at[pl.ds(r0, CPY)], sout.at[pl.ds(r0, CPY)])
        pltpu.sync_copy(z1_hbm.at[pl.ds(d0, DCPY)], sden.at[pl.ds(d0, DCPY)])

        @pl.when(s == 0)
        def _zero_tail():
            t0 = NS * CPY
            pltpu.sync_copy(z64_hbm.at[pl.ds(t0, TAIL)],
                            sout.at[pl.ds(t0, TAIL)])

        plsc.subcore_barrier()
        mvec = m_v[...]

        @pl.loop(0, NGRP)
        def _group(b):
            gather = pltpu.async_copy(h_hbm.at[src_v.at[b]], rows_v, gsem)
            for g in range(GRP * STRIP // 16):
                sl = pl.ds(g * 16, 16)
                sidx = src_v[b, sl]
                didx = dst_v[b, sl]
                lg = (plsc.load_gather(es_v, [sidx])
                      + plsc.load_gather(ed_v, [didx]))
                lg = jnp.maximum(lg, 0.2 * lg)
                ex_v[sl] = jnp.exp(lg - mvec)
            if want_ex:
                pltpu.sync_copy(ex_v, ex_hbm.at[wid].at[b])
            den_sc = pltpu.async_copy(ex_v, sden.at[dst_v.at[b]],
                                      ssem, add=True)
            gather.wait()

            @pl.loop(0, GRP * STRIP // 16)
            def _blk(g):
                exv = ex_v[pl.ds(g * 16, 16)]
                for r in range(16):
                    e = exv[r]
                    i = g * 16 + r
                    for q in range(F // 16):
                        sl = pl.ds(q * 16, 16)
                        rows_v[i, sl] = rows_v[i, sl] * e

            agg_sc = pltpu.async_copy(rows_v, sout.at[dst_v.at[b]],
                                      ssem, add=True)
            den_sc.wait()
            agg_sc.wait()

        plsc.subcore_barrier()
        pltpu.sync_copy(sout.at[pl.ds(r0, CPY)],
                        agg_hbm.at[c].at[pl.ds(r0, CPY)])
        pltpu.sync_copy(sden.at[pl.ds(d0, DCPY)],
                        den_hbm.at[c].at[pl.ds(d0, DCPY)])

        @pl.when(s == 0)
        def _out_tail():
            t0 = NS * CPY
            pltpu.sync_copy(sout.at[pl.ds(t0, TAIL)],
                            agg_hbm.at[c].at[pl.ds(t0, TAIL)])

    return pl.kernel(body, out_type=outs, mesh=mesh, scratch_types=scratch,
                     compiler_params=_SC_PARAMS)


_sc_edge_first = _make_sc_edge(want_ex=True)
_sc_edge = _make_sc_edge(want_ex=False)


def _alpha_body(ex_hbm, dstf_hbm, den_hbm, al_hbm,
                den_v, den2_v, ex_v, dst_v, al_v):
    c = lax.axis_index("c")
    s = lax.axis_index("s")
    wid = c * NS + s
    pltpu.sync_copy(den_hbm.at[0], den_v)
    pltpu.sync_copy(den_hbm.at[1], den2_v)
    pltpu.sync_copy(ex_hbm.at[wid], ex_v)
    pltpu.sync_copy(dstf_hbm.at[wid], dst_v)

    @pl.loop(0, NP // 16)
    def _sum(i):
        sl = pl.ds(i * 16, 16)
        den_v[sl] = den_v[sl] + den2_v[sl]

    @pl.loop(0, EPW // 16)
    def _alpha(g):
        sl = pl.ds(g * 16, 16)
        d = plsc.load_gather(den_v, [dst_v[sl]])
        al_v[sl] = ex_v[sl] / (d + 1e-30)

    pltpu.sync_copy(al_v, al_hbm.at[wid])


_sc_alpha = pl.kernel(
    _alpha_body,
    out_type=jax.ShapeDtypeStruct((NW, EPW), jnp.float32),
    mesh=plsc.VectorSubcoreMesh(core_axis_name="c", subcore_axis_name="s"),
    scratch_types=[
        pltpu.VMEM((NP,), jnp.float32),
        pltpu.VMEM((NP,), jnp.float32),
        pltpu.VMEM((EPW,), jnp.float32),
        pltpu.VMEM((EPW,), jnp.int32),
        pltpu.VMEM((EPW,), jnp.float32),
    ],
    compiler_params=_SC_PARAMS,
)


def kernel(X_in, A_in, E_in,
           W1, as1, ad1, b1,
           W2, as2, ad2, b2,
           W3, as3, ad3, b3,
           W4, as4, ad4, b4,
           W5, as5, ad5, b5,
           fc_W, fc_b, out_W, out_b):
    del E_in
    src3 = A_in[0].reshape(NW, NGRP, GRP * STRIP)
    dst3 = A_in[1].reshape(NW, NGRP, GRP * STRIP)
    dstf = A_in[1].reshape(NW, EPW)
    z64 = jnp.zeros((N, F), jnp.float32)
    z1 = jnp.zeros((NP,), jnp.float32)

    layers = [(W1, as1, ad1, b1), (W2, as2, ad2, b2), (W3, as3, ad3, b3),
              (W4, as4, ad4, b4), (W5, as5, ad5, b5)]

    agg = den = ex1 = den1 = None
    for i, (W, a_s, a_d, b) in enumerate(layers):
        if i == 0:
            h, es, ed, m = _tc_first(X_in, W, a_s.reshape(1, F),
                                     a_d.reshape(1, F))
        else:
            b_prev = layers[i - 1][3]
            h, es, ed, m = _tc_mid(agg, den.reshape(NC, NP, 1),
                                   b_prev.reshape(1, F), W,
                                   a_s.reshape(1, F), a_d.reshape(1, F))
        args = (h, es.reshape(N), ed.reshape(N), src3, dst3, m, z64, z1)
        if i == 0:
            agg, den, ex1 = _sc_edge_first(*args)
            den1 = den
        else:
            agg, den = _sc_edge(*args)

    alpha = _sc_alpha(ex1.reshape(NW, EPW), dstf, den1)

    out = _tc_head(agg, den.reshape(NC, NP, 1), b5.reshape(1, F),
                   fc_W, fc_b.reshape(1, 32), out_W, out_b.reshape(1, 1))
    return out.reshape(1), alpha.reshape(E)


# speedup vs baseline: 1.0947x; 1.0947x over previous
import dataclasses

import jax
import jax.numpy as jnp
from jax import lax
from jax.experimental import pallas as pl
from jax.experimental.pallas import tpu as pltpu
from jax.experimental.pallas import tpu_sc as plsc

_SC_PARAMS = pltpu.CompilerParams()
if "needs_layout_passes" in pltpu.CompilerParams.__dataclass_fields__:
    _SC_PARAMS = dataclasses.replace(_SC_PARAMS, needs_layout_passes=False)
if "use_tc_tiling_on_sc" in pltpu.CompilerParams.__dataclass_fields__:
    _SC_PARAMS = dataclasses.replace(_SC_PARAMS, use_tc_tiling_on_sc=False)

N = 10000
E = 320000
D = 128
F = 64

NC = 2
NS = 16
NW = NC * NS
EPW = E // NW
STRIP = 80
NSTR = EPW // STRIP
GRP = 5
NGRP = NSTR // GRP
CPY = 624
TAIL = N - NS * CPY
NP = 10240
DCPY = NP // NS


def _tc_first_body(x_ref, w_ref, as_ref, ad_ref, h_ref, es_ref, ed_ref, m_ref):
    h = jnp.dot(x_ref[...], w_ref[...], preferred_element_type=jnp.float32)
    h_ref[...] = h
    es = jnp.sum(h * as_ref[...], axis=1, keepdims=True)
    ed = jnp.sum(h * ad_ref[...], axis=1, keepdims=True)
    es_ref[...] = es
    ed_ref[...] = ed
    m = jnp.maximum(jnp.max(es) + jnp.max(ed), 0.0)
    m_ref[...] = jnp.full((8, 128), m, jnp.float32)


def _tc_mid_body(agg_ref, den_ref, b_ref, w_ref, as_ref, ad_ref,
                 h_ref, es_ref, ed_ref, m_ref):
    a = agg_ref[0] + agg_ref[1]
    dn = den_ref[0, :N] + den_ref[1, :N]
    x = jnp.maximum(a / (dn + 1e-30) + b_ref[...], 0.0)
    h = jnp.dot(x, w_ref[...], preferred_element_type=jnp.float32)
    h_ref[...] = h
    es = jnp.sum(h * as_ref[...], axis=1, keepdims=True)
    ed = jnp.sum(h * ad_ref[...], axis=1, keepdims=True)
    es_ref[...] = es
    ed_ref[...] = ed
    m = jnp.maximum(jnp.max(es) + jnp.max(ed), 0.0)
    m_ref[...] = jnp.full((8, 128), m, jnp.float32)


def _tc_head_body(agg_ref, den_ref, b_ref, fcw_ref, fcb_ref, ow_ref, ob_ref,
                  out_ref):
    a = agg_ref[0] + agg_ref[1]
    dn = den_ref[0, :N] + den_ref[1, :N]
    x = jnp.maximum(a / (dn + 1e-30) + b_ref[...], 0.0)
    pooled = jnp.mean(x, axis=0, keepdims=True)
    hfc = jnp.maximum(
        jnp.dot(pooled, fcw_ref[...], preferred_element_type=jnp.float32)
        + fcb_ref[...], 0.0)
    out_ref[...] = jax.nn.sigmoid(
        jnp.dot(hfc, ow_ref[...], preferred_element_type=jnp.float32)
        + ob_ref[...])


_LAYER_OUT = [
    jax.ShapeDtypeStruct((N, F), jnp.float32),
    jax.ShapeDtypeStruct((N, 1), jnp.float32),
    jax.ShapeDtypeStruct((N, 1), jnp.float32),
    jax.ShapeDtypeStruct((8, 128), jnp.float32),
]

_tc_first = pl.pallas_call(_tc_first_body, out_shape=_LAYER_OUT)
_tc_mid = pl.pallas_call(_tc_mid_body, out_shape=_LAYER_OUT)
_tc_head = pl.pallas_call(
    _tc_head_body, out_shape=jax.ShapeDtypeStruct((1, 1), jnp.float32))


def _make_sc_edge(want_ex):
    outs = [
        jax.ShapeDtypeStruct((NC, N, F), jnp.float32),
        jax.ShapeDtypeStruct((NC, NP), jnp.float32),
    ]
    if want_ex:
        outs.append(jax.ShapeDtypeStruct((NW, NGRP, GRP, STRIP), jnp.float32))
    scratch = [
        pltpu.VMEM((N,), jnp.float32),
        pltpu.VMEM((N,), jnp.float32),
        pltpu.VMEM((NSTR, STRIP), jnp.int32),
        pltpu.VMEM((NSTR, STRIP), jnp.int32),
        pltpu.VMEM((16,), jnp.float32),
        pltpu.VMEM((GRP, STRIP), jnp.float32),
        pltpu.VMEM((GRP, STRIP, F), jnp.float32),
        pltpu.VMEM_SHARED((N, F), jnp.float32),
        pltpu.VMEM_SHARED((NP,), jnp.float32),
    ] + [pltpu.SemaphoreType.DMA] * 2
    mesh = plsc.VectorSubcoreMesh(core_axis_name="c", subcore_axis_name="s")

    def body(h_hbm, es_hbm, ed_hbm, src_hbm, dst_hbm, m_hbm, z64_hbm, z1_hbm,
             agg_hbm, den_hbm, *rest):
        if want_ex:
            ex_hbm, *rest = rest
        (es_v, ed_v, src_v, dst_v, m_v, ex_v, rows_v, sout, sden,
         gsem, ssem) = rest
        c = lax.axis_index("c")
        s = lax.axis_index("s")
        wid = c * NS + s
        pltpu.sync_copy(es_hbm, es_v)
        pltpu.sync_copy(ed_hbm, ed_v)
        pltpu.sync_copy(src_hbm.at[wid], src_v)
        pltpu.sync_copy(dst_hbm.at[wid], dst_v)
        pltpu.sync_copy(m_hbm.at[0, pl.ds(0, 16)], m_v)
        r0 = s * CPY
        d0 = s * DCPY
        pltpu.sync_copy(z64_hbm.at[pl.ds(r0, CPY)], sout.at[pl.ds(r0, CPY)])
        pltpu.sync_copy(z1_hbm.at[pl.ds(d0, DCPY)], sden.at[pl.ds(d0, DCPY)])

        @pl.when(s == 0)
        def _zero_tail():
            t0 = NS * CPY
            pltpu.sync_copy(z64_hbm.at[pl.ds(t0, TAIL)],
                            sout.at[pl.ds(t0, TAIL)])

        plsc.subcore_barrier()
        mvec = m_v[...]

        @pl.loop(0, NGRP)
        def _group(b):
            gathers = [
                pltpu.async_copy(h_hbm.at[src_v.at[b * GRP + k]],
                                 rows_v.at[k], gsem)
                for k in range(GRP)
            ]
            for k in range(GRP):
                for g in range(STRIP // 16):
                    sl = pl.ds(g * 16, 16)
                    sidx = src_v[b * GRP + k, sl]
                    didx = dst_v[b * GRP + k, sl]
                    lg = (plsc.load_gather(es_v, [sidx])
                          + plsc.load_gather(ed_v, [didx]))
                    lg = jnp.maximum(lg, 0.2 * lg)
                    ex_v[k, sl] = jnp.exp(lg - mvec)
            if want_ex:
                pltpu.sync_copy(ex_v, ex_hbm.at[wid].at[b])
            scatters = [
                pltpu.async_copy(ex_v.at[k], sden.at[dst_v.at[b * GRP + k]],
                                 ssem, add=True)
                for k in range(GRP)
            ]
            for hnd in gathers:
                hnd.wait()
            for k in range(GRP):
                @pl.loop(0, STRIP // 16)
                def _blk(g, k=k):
                    exv = ex_v[k, pl.ds(g * 16, 16)]
                    for r in range(16):
                        e = exv[r]
                        i = g * 16 + r
                        for q in range(F // 16):
                            sl = pl.ds(q * 16, 16)
                            rows_v[k, i, sl] = rows_v[k, i, sl] * e
            scatters += [
                pltpu.async_copy(rows_v.at[k], sout.at[dst_v.at[b * GRP + k]],
                                 ssem, add=True)
                for k in range(GRP)
            ]
            for hnd in scatters:
                hnd.wait()

        plsc.subcore_barrier()
        pltpu.sync_copy(sout.at[pl.ds(r0, CPY)],
                        agg_hbm.at[c].at[pl.ds(r0, CPY)])
        pltpu.sync_copy(sden.at[pl.ds(d0, DCPY)],
                        den_hbm.at[c].at[pl.ds(d0, DCPY)])

        @pl.when(s == 0)
        def _out_tail():
            t0 = NS * CPY
            pltpu.sync_copy(sout.at[pl.ds(t0, TAIL)],
                            agg_hbm.at[c].at[pl.ds(t0, TAIL)])

    return pl.kernel(body, out_type=outs, mesh=mesh, scratch_types=scratch,
                     compiler_params=_SC_PARAMS)


_sc_edge_first = _make_sc_edge(want_ex=True)
_sc_edge = _make_sc_edge(want_ex=False)


def _alpha_body(ex_hbm, dstf_hbm, den_hbm, al_hbm,
                den_v, den2_v, ex_v, dst_v, al_v):
    c = lax.axis_index("c")
    s = lax.axis_index("s")
    wid = c * NS + s
    pltpu.sync_copy(den_hbm.at[0], den_v)
    pltpu.sync_copy(den_hbm.at[1], den2_v)
    pltpu.sync_copy(ex_hbm.at[wid], ex_v)
    pltpu.sync_copy(dstf_hbm.at[wid], dst_v)

    @pl.loop(0, NP // 16)
    def _sum(i):
        sl = pl.ds(i * 16, 16)
        den_v[sl] = den_v[sl] + den2_v[sl]

    @pl.loop(0, EPW // 16)
    def _alpha(g):
        sl = pl.ds(g * 16, 16)
        d = plsc.load_gather(den_v, [dst_v[sl]])
        al_v[sl] = ex_v[sl] / (d + 1e-30)

    pltpu.sync_copy(al_v, al_hbm.at[wid])


_sc_alpha = pl.kernel(
    _alpha_body,
    out_type=jax.ShapeDtypeStruct((NW, EPW), jnp.float32),
    mesh=plsc.VectorSubcoreMesh(core_axis_name="c", subcore_axis_name="s"),
    scratch_types=[
        pltpu.VMEM((NP,), jnp.float32),
        pltpu.VMEM((NP,), jnp.float32),
        pltpu.VMEM((EPW,), jnp.float32),
        pltpu.VMEM((EPW,), jnp.int32),
        pltpu.VMEM((EPW,), jnp.float32),
    ],
    compiler_params=_SC_PARAMS,
)


def kernel(X_in, A_in, E_in,
           W1, as1, ad1, b1,
           W2, as2, ad2, b2,
           W3, as3, ad3, b3,
           W4, as4, ad4, b4,
           W5, as5, ad5, b5,
           fc_W, fc_b, out_W, out_b):
    del E_in
    src3 = A_in[0].reshape(NW, NSTR, STRIP)
    dst3 = A_in[1].reshape(NW, NSTR, STRIP)
    dstf = A_in[1].reshape(NW, EPW)
    z64 = jnp.zeros((N, F), jnp.float32)
    z1 = jnp.zeros((NP,), jnp.float32)

    layers = [(W1, as1, ad1, b1), (W2, as2, ad2, b2), (W3, as3, ad3, b3),
              (W4, as4, ad4, b4), (W5, as5, ad5, b5)]

    agg = den = ex1 = den1 = None
    for i, (W, a_s, a_d, b) in enumerate(layers):
        if i == 0:
            h, es, ed, m = _tc_first(X_in, W, a_s.reshape(1, F),
                                     a_d.reshape(1, F))
        else:
            b_prev = layers[i - 1][3]
            h, es, ed, m = _tc_mid(agg, den.reshape(NC, NP, 1),
                                   b_prev.reshape(1, F), W,
                                   a_s.reshape(1, F), a_d.reshape(1, F))
        args = (h, es.reshape(N), ed.reshape(N), src3, dst3, m, z64, z1)
        if i == 0:
            agg, den, ex1 = _sc_edge_first(*args)
            den1 = den
        else:
            agg, den = _sc_edge(*args)

    alpha = _sc_alpha(ex1.reshape(NW, EPW), dstf, den1)

    out = _tc_head(agg, den.reshape(NC, NP, 1), b5.reshape(1, F),
                   fc_W, fc_b.reshape(1, 32), out_W, out_b.reshape(1, 1))
    return out.reshape(1), alpha.reshape(E)
